# baseline (speedup 1.0000x reference)
import numpy as np
import jax
import jax.numpy as jnp
from jax import lax
from jax.experimental import pallas as pl
from jax.experimental.pallas import tpu as pltpu
from jax.experimental.pallas import tpu_sc as plsc

_BATCH = 16384
_DIM = 64
_NTAB = 4
_V = 100000
_ALIGNED = 99968
_NC, _NS, _NW = 2, 16, 32
_BPW = _BATCH // _NW

_W = 768
_NBANDS = _W // 128
_NRANGE = 8
_RANGEW = 16 * _W

_SPT = _BATCH + 8
_ST_DUMP = _NTAB * _SPT
_ST_ROWS = _ST_DUMP + 16

_T_SLOTS = 13


def _range_bounds(c, r):
    if c == 3:
        return (0, 1024) if r == 0 else (0, 0)
    lo = r * _RANGEW
    hi = _V if r == _NRANGE - 1 else (r + 1) * _RANGEW
    return lo, hi


def _build_desc():
    big_starts = [k * _W for k in range(_ALIGNED // _W)] + [_ALIGNED - _W]
    tasks_sc = ([], [])
    for c in range(3):
        for st in big_starts:
            r = min(st // _RANGEW, _NRANGE - 1)
            tasks_sc[0 if r < 4 else 1].append((c, st, r, 1))
    for st in (0, 256):
        tasks_sc[0].append((3, st, 0, 1))
    desc = np.zeros((_NW, _T_SLOTS, 16), np.int32)
    for sc in (0, 1):
        assert len(tasks_sc[sc]) <= _T_SLOTS * _NS
        for t, task in enumerate(tasks_sc[sc]):
            w = sc * _NS + (t % _NS)
            desc[w, t // _NS, :4] = task
    return desc

_DESC = _build_desc()


def _k1_body(uh, rh, ih, nh, t0, t1, t2, t3, tl0, tl1, tl2, desc_h, st_out,
             chunk3, piece, clist, plist, f0, f1, tails_v, cnts_v, pcnt_v,
             desc_v, slists, scounts, gsem, s0, s1):
    idx_hbms = (uh, rh, ih, nh)
    tabs = (t0, t1, t2, t3)
    tails_h = (tl0, tl1, tl2)
    sc = lax.axis_index("c")
    s = lax.axis_index("s")
    wid = s * _NC + sc
    iota = lax.iota(jnp.int32, 16)
    fsems = (s0, s1)
    frags = (f0, f1)

    def drain_frag(half):
        pltpu.make_async_copy(
            st_out.at[pl.ds(0, 16), :], frags[half], fsems[half]).wait()

    for tc in range(3):
        pltpu.sync_copy(tails_h[tc], tails_v.at[pl.ds(tc * 2048, 2048)])
    pltpu.sync_copy(desc_h.at[wid], desc_v)

    r_mine = s % _NRANGE
    h_mine = s // _NRANGE
    ibase0 = h_mine * (_BATCH // 2)
    cvec = iota * 0
    for c in range(_NTAB):
        lo_v = jnp.int32(0)
        hi_v = jnp.int32(0)
        for rr in range(_NRANGE):
            lo_s, hi_s = _range_bounds(c, rr)
            lo_v = jnp.where(r_mine == rr, jnp.int32(lo_s), lo_v)
            hi_v = jnp.where(r_mine == rr, jnp.int32(hi_s), hi_v)

        def piece_body(p, cur, c=c, lo_v=lo_v, hi_v=hi_v):
            pltpu.sync_copy(
                idx_hbms[c].at[pl.ds(ibase0 + p * 2048, 2048)], piece)

            def grp(g, cur2):
                jv = piece[pl.ds(g * 16, 16)]
                m = (jv >= lo_v) & (jv < hi_v)
                q = jv - lo_v
                e = (ibase0 + p * 2048 + g * 16 + iota) * jnp.int32(16384) + q
                cs = plsc.cumsum(m.astype(jnp.int32))
                plsc.store_scatter(plist, [cur2 + cs - 1], e, mask=m)
                return cur2 + cs[15]
            return lax.fori_loop(0, 128, grp, cur)
        cnt = lax.fori_loop(0, 4, piece_body, jnp.int32(0))
        pltpu.sync_copy(plist, slists.at[c, r_mine, h_mine])
        cvec = jnp.where(iota == c, cnt, cvec)
    pcnt_v[...] = cvec
    pltpu.sync_copy(pcnt_v, scounts.at[pl.ds((r_mine * 2 + h_mine) * 16, 16)])
    plsc.subcore_barrier()
    pltpu.sync_copy(scounts, cnts_v)

    def read_cnt(flat_idx):
        return plsc.load_gather(cnts_v, [iota * 0 + flat_idx])[0]

    def extract_groups(ncl, tab_v, get_vals):
        npairs = (ncl + 31) // 32

        def pair(t, _):
            for half in range(2):
                g = t * 2 + half

                @pl.when(t > 0)
                def _(half=half):
                    drain_frag(half)
                frag = frags[half]
                ev = clist[pl.ds(g * 16, 16)]
                m = (g * 16 + iota) < ncl
                q = jnp.where(m, ev & jnp.int32(2047), 0)
                i_v = jnp.where(m, lax.shift_right_logical(ev, 11), 0)
                get_vals(q, m, frag)
                dst = jnp.where(m, tab_v * jnp.int32(_SPT) + i_v,
                                jnp.int32(_ST_DUMP) + iota)
                pltpu.async_copy(frag, st_out.at[dst], fsems[half])
            return 0
        lax.fori_loop(0, npairs, pair, 0)

        @pl.when(npairs > 0)
        def _():
            drain_frag(0)
            drain_frag(1)

    def slot_body(slot, _carry):
        dv = desc_v[pl.ds(slot * 16, 16)]
        tab_v = dv[0]
        start_v = dv[1]
        rsel_v = dv[2]

        for c in range(_NTAB):
            @pl.when(tab_v == c)
            def _(c=c):
                for b in range(_NBANDS):
                    off = pl.multiple_of(start_v + b * 128, 128)
                    pltpu.async_copy(
                        tabs[c].at[:, pl.ds(off, 128)],
                        chunk3.at[b], gsem)

        rbase_v = jnp.int32(0)
        for c in range(_NTAB):
            for rr in range(_NRANGE):
                lo_s, _ = _range_bounds(c, rr)
                rbase_v = jnp.where((tab_v == c) & (rsel_v == rr),
                                    jnp.int32(lo_s), rbase_v)
        delta = start_v - rbase_v

        ncl = jnp.int32(0)
        for h in range(2):
            cnt = read_cnt((rsel_v * 2 + h) * 16 + tab_v)

            def pc_body(p, ncl2, h=h, cnt=cnt):
                pltpu.sync_copy(
                    slists.at[tab_v, rsel_v, h, pl.ds(p * 2048, 2048)], piece)

                def grp(g, ncl3):
                    base = p * 2048 + g * 16
                    ev = piece[pl.ds(g * 16, 16)]
                    valid = (base + iota) < cnt
                    qc = (ev & jnp.int32(16383)) - delta
                    m = valid & (qc >= 0) & (qc < _W)
                    e2 = lax.shift_right_logical(ev, 14) * jnp.int32(2048) + qc
                    cs = plsc.cumsum(m.astype(jnp.int32))
                    plsc.store_scatter(clist, [ncl3 + cs - 1], e2, mask=m)
                    return ncl3 + cs[15]
                return lax.fori_loop(0, 128, grp, ncl2)
            ncl = lax.fori_loop(0, (cnt + 2047) // 2048, pc_body, ncl)

        def drain_band(_b, x):
            pltpu.make_async_copy(
                tabs[0].at[:, pl.ds(0, 128)], chunk3.at[0], gsem).wait()
            return x
        lax.fori_loop(0, _NBANDS, drain_band, 0)

        def chunk_vals(q, m, frag):
            band = lax.shift_right_logical(q, 7)
            lane = q & jnp.int32(127)
            for r in range(_DIM):
                v = plsc.load_gather(chunk3, [band, iota * 0 + r, lane],
                                     mask=m)
                plsc.store_scatter(frag, [iota, iota * 0 + r], v, mask=m)
        extract_groups(ncl, tab_v, chunk_vals)
        return 0

    lax.fori_loop(0, _T_SLOTS, slot_body, 0)

    ibase = wid * _BPW
    for tc in range(3):
        pltpu.sync_copy(idx_hbms[tc].at[pl.ds(ibase, _BPW)],
                        piece.at[pl.ds(0, _BPW)])

        def tail_grp(g, _carry, tc=tc):
            jv = piece[pl.ds(g * 16, 16)]
            m = jv >= _ALIGNED
            npos = plsc.all_reduce_population_count(m)

            @pl.when(npos[0] > 0)
            def _():
                q = jnp.where(m, jv - _ALIGNED, 0)
                i_v = ibase + g * 16 + iota
                frag = frags[0]
                for r in range(_DIM):
                    v = plsc.load_gather(
                        tails_v, [tc * 2048 + q * _DIM + r], mask=m)
                    plsc.store_scatter(frag, [iota, iota * 0 + r], v, mask=m)
                dst = jnp.where(m, tc * _SPT + i_v,
                                jnp.int32(_ST_DUMP) + iota)
                pltpu.async_copy(frag, st_out.at[dst], fsems[0]).wait()
            return 0
        lax.fori_loop(0, _BPW // 16, tail_grp, 0)


def _k2_body(st_h, out_h, bufs, stags, gsems, ssems):
    sc = lax.axis_index("c")
    s = lax.axis_index("s")
    row0 = (s * _NC + sc) * _BPW
    hs = [None, None]
    hg = [[None] * _NTAB, [None] * _NTAB]

    def fire_loads(j, pb):
        for c in range(_NTAB):
            hg[pb][c] = pltpu.async_copy(
                st_h.at[pl.ds(c * _SPT + row0 + j * 64, 64), :],
                bufs[pb][c], gsems[pb])

    def assemble_store(j, pb):
        for c in range(_NTAB):
            hg[pb][c].wait()
        stag = stags[pb]

        def rb(r, _):
            for c in range(_NTAB):
                for k in range(_DIM // 16):
                    stag[r, pl.ds(c * _DIM + k * 16, 16)] = (
                        bufs[pb][c][r, pl.ds(k * 16, 16)])
            return 0
        lax.fori_loop(0, 64, rb, 0)
        if hs[pb] is not None:
            hs[pb].wait()
        hs[pb] = pltpu.async_copy(
            stag, out_h.at[pl.ds(row0 + j * 64, 64), :], ssems[pb])

    fire_loads(0, 0)
    for j in range(_BPW // 64):
        pb = j % 2
        if j + 1 < _BPW // 64:
            fire_loads(j + 1, 1 - pb)
        assemble_store(j, pb)
    hs[0].wait()
    hs[1].wait()


def _run(uid, rid, ing, nut, t0T, t1T, t2T, t3T, tl0, tl1, tl2, desc):
    k1 = pl.kernel(
        _k1_body,
        out_type=jax.ShapeDtypeStruct((_ST_ROWS, 128), jnp.float32),
        mesh=plsc.VectorSubcoreMesh(core_axis_name="c", subcore_axis_name="s"),
        compiler_params=pltpu.CompilerParams(needs_layout_passes=False),
        scratch_types=[
            pltpu.VMEM((_NBANDS, 64, 128), jnp.float32),
            pltpu.VMEM((2048,), jnp.int32),
            pltpu.VMEM((16384,), jnp.int32),
            pltpu.VMEM((8192,), jnp.int32),
            pltpu.VMEM((16, 128), jnp.float32),
            pltpu.VMEM((16, 128), jnp.float32),
            pltpu.VMEM((3 * 2048,), jnp.float32),
            pltpu.VMEM((256,), jnp.int32),
            pltpu.VMEM((16,), jnp.int32),
            pltpu.VMEM((_T_SLOTS * 16,), jnp.int32),
            pltpu.VMEM_SHARED((_NTAB, _NRANGE, 2, 8192), jnp.int32),
            pltpu.VMEM_SHARED((256,), jnp.int32),
            pltpu.SemaphoreType.DMA,
            pltpu.SemaphoreType.DMA,
            pltpu.SemaphoreType.DMA,
        ],
    )
    st = k1(uid, rid, ing, nut, t0T, t1T, t2T, t3T, tl0, tl1, tl2, desc)

    def k2_wrap(st_h, out_h, b00, b01, b02, b03, b10, b11, b12, b13,
                st0, st1, g0, g1, e0, e1):
        _k2_body(st_h, out_h,
                 ((b00, b01, b02, b03), (b10, b11, b12, b13)),
                 (st0, st1), (g0, g1), (e0, e1))

    k2 = pl.kernel(
        k2_wrap,
        out_type=jax.ShapeDtypeStruct((_BATCH, _NTAB * _DIM), jnp.float32),
        mesh=plsc.VectorSubcoreMesh(core_axis_name="c", subcore_axis_name="s"),
        compiler_params=pltpu.CompilerParams(needs_layout_passes=False),
        scratch_types=(
            [pltpu.VMEM((64, 128), jnp.float32)] * 8
            + [pltpu.VMEM((64, 256), jnp.float32)] * 2
            + [pltpu.SemaphoreType.DMA] * 4
        ),
    )
    return k2(st)


def kernel(uid, rid, ing, nut, user_table, recipe_table, ingredient_table,
           nutrition_table):
    uid = uid.astype(jnp.int32)
    rid = rid.astype(jnp.int32)
    ing = ing.astype(jnp.int32)
    nut = nut.astype(jnp.int32)
    t0 = user_table.astype(jnp.float32)
    t1 = recipe_table.astype(jnp.float32)
    t2 = ingredient_table.astype(jnp.float32)
    t3 = nutrition_table.astype(jnp.float32)
    t0T, t1T, t2T = t0.T, t1.T, t2.T
    t3T = jnp.pad(t3, ((0, 1024 - t3.shape[0]), (0, 0))).T
    tl0 = t0[_ALIGNED:].reshape(-1)
    tl1 = t1[_ALIGNED:].reshape(-1)
    tl2 = t2[_ALIGNED:].reshape(-1)
    desc = jnp.asarray(_DESC.reshape(_NW, -1))
    return _run(uid, rid, ing, nut, t0T, t1T, t2T, t3T, tl0, tl1, tl2, desc)


# speedup vs baseline: 3.5265x; 3.5265x over previous
import jax
import jax.numpy as jnp
from jax import lax
from jax.experimental import pallas as pl
from jax.experimental.pallas import tpu as pltpu
from jax.experimental.pallas import tpu_sc as plsc

_BATCH = 16384
_DIM = 64
_NTAB = 4
_NC = 2
_NS = 16
_NW = _NC * _NS
_BPW = _BATCH // _NW
_CHUNK = 128
_NCHUNK = _BPW // _CHUNK
_NCHUNKS_TOTAL = _NTAB * _NCHUNK
_NBUF = 4
_LAG = 2


def _body(uid_h, rid_h, ing_h, nut_h, user_t, recipe_t, ingredient_t,
          nutrition_t, out_hbm, idx_v, bufs, gsems, ssems):
    tables = (user_t, recipe_t, ingredient_t, nutrition_t)
    idx_hbms = (uid_h, rid_h, ing_h, nut_h)
    wid = lax.axis_index("s") * _NC + lax.axis_index("c")
    base = wid * _BPW

    for c in range(_NTAB):
        pltpu.sync_copy(idx_hbms[c].at[pl.ds(base, _BPW)], idx_v.at[c])

    chunks = [(c, j) for c in range(_NTAB) for j in range(_NCHUNK)]
    hg = [None] * _NCHUNKS_TOTAL
    hs = [None] * _NCHUNKS_TOTAL

    def fire_store(i):
        c, j = chunks[i]
        k = i % _NBUF
        hg[i].wait()
        hs[i] = pltpu.async_copy(
            bufs[k],
            out_hbm.at[pl.ds(base + j * _CHUNK, _CHUNK),
                       pl.ds(c * _DIM, _DIM)],
            ssems[k])

    for i, (c, j) in enumerate(chunks):
        k = i % _NBUF
        if i >= _NBUF:
            hs[i - _NBUF].wait()
        hg[i] = pltpu.async_copy(
            tables[c].at[idx_v.at[c, pl.ds(j * _CHUNK, _CHUNK)]],
            bufs[k], gsems[k])
        if i >= _LAG:
            fire_store(i - _LAG)
    for i in range(_NCHUNKS_TOTAL - _LAG, _NCHUNKS_TOTAL):
        fire_store(i)
    for i in range(_NCHUNKS_TOTAL - _NBUF, _NCHUNKS_TOTAL):
        hs[i].wait()


def _sc_call(uid, rid, ing, nut, user_t, recipe_t, ingredient_t, nutrition_t):
    def body(uh, rh, ih, nh, ut, rt, it, nt, out_hbm, idx_v, b0, b1, b2, b3,
             g0, g1, g2, g3, s0, s1, s2, s3):
        _body(uh, rh, ih, nh, ut, rt, it, nt, out_hbm, idx_v,
              (b0, b1, b2, b3), (g0, g1, g2, g3), (s0, s1, s2, s3))

    f = pl.kernel(
        body,
        out_type=jax.ShapeDtypeStruct((_BATCH, _NTAB * _DIM), jnp.float32),
        mesh=plsc.VectorSubcoreMesh(core_axis_name="c", subcore_axis_name="s"),
        scratch_types=[
            pltpu.VMEM((_NTAB, _BPW), jnp.int32),
        ] + [pltpu.VMEM((_CHUNK, _DIM), jnp.float32)] * _NBUF
          + [pltpu.SemaphoreType.DMA] * (2 * _NBUF),
        compiler_params=pltpu.CompilerParams(use_tc_tiling_on_sc=False),
    )
    return f(uid, rid, ing, nut, user_t, recipe_t, ingredient_t, nutrition_t)


def kernel(uid, rid, ing, nut, user_table, recipe_table, ingredient_table,
           nutrition_table):
    return _sc_call(uid.astype(jnp.int32), rid.astype(jnp.int32),
                    ing.astype(jnp.int32), nut.astype(jnp.int32),
                    user_table.astype(jnp.float32),
                    recipe_table.astype(jnp.float32),
                    ingredient_table.astype(jnp.float32),
                    nutrition_table.astype(jnp.float32))
